# baseline (speedup 1.0000x reference)
import jax
import jax.numpy as jnp
from jax.experimental import pallas as pl
from jax.experimental.pallas import tpu as pltpu

_B = 4096
_R = 64
_NC = 8
_TILE = 64
_SUB = 8
_STEP = _TILE * _SUB
_NT = _B // _STEP

_BMM_DIMS = (((2,), (1,)), ((0,), (0,)))


def _tr_kernel(idx_ref, c0_ref, c1_ref, c2_ref, c3_ref, c4_ref, c5_ref,
               c6_ref, c7_ref, out_ref,
               tbl, tblt, g0, g1, g2, g3, g4, g5, g6, g7, s0, s1, sf,
               norm, lognorm):
    @pl.when(pl.program_id(0) == 0)
    def _build_tables():
        crefs = [c0_ref, c1_ref, c2_ref, c3_ref, c4_ref, c5_ref, c6_ref,
                 c7_ref]
        for i in range(_NC):
            sf[...] = jax.nn.softplus(crefs[i][...])
            if i < 4:
                tbl[pl.ds(_R * i, _R)] = sf[...].astype(jnp.bfloat16)
            else:
                tblt[pl.ds(_R * (i - 4), _R)] = jnp.swapaxes(
                    sf[...], 1, 2).astype(jnp.bfloat16)
            s = jnp.sum(sf[...], axis=0)
            if i == 0:
                norm[...] = s
            else:
                norm[...] = norm[...] @ s
        eye = (jax.lax.broadcasted_iota(jnp.int32, (_R, _R), 0)
               == jax.lax.broadcasted_iota(jnp.int32, (_R, _R), 1))
        tr_n = jnp.sum(jnp.where(eye, norm[...], 0.0))
        lognorm[0, 0] = jnp.log(tr_n)

    def bmm(x, y):
        return jax.lax.dot_general(x, y, _BMM_DIMS,
                                   preferred_element_type=jnp.float32)

    for sub in range(_SUB):
        base = sub * _TILE

        def gather_body(b, carry):
            r = base + b
            g0[b] = tbl[idx_ref[r, 0]]
            g1[b] = tbl[idx_ref[r, 1]]
            g2[b] = tbl[idx_ref[r, 2]]
            g3[b] = tbl[idx_ref[r, 3]]
            g4[b] = tblt[idx_ref[r, 4]]
            g5[b] = tblt[idx_ref[r, 5]]
            g6[b] = tblt[idx_ref[r, 6]]
            g7[b] = tblt[idx_ref[r, 7]]
            return carry

        jax.lax.fori_loop(0, _TILE, gather_body, 0, unroll=8)

        bf = jnp.bfloat16
        s0[...] = bmm(g0[...], g1[...]).astype(bf)
        s1[...] = bmm(g2[...], g3[...]).astype(bf)
        g2[...] = bmm(g7[...], g6[...]).astype(bf)
        g3[...] = bmm(g5[...], g4[...]).astype(bf)
        a = bmm(s0[...], s1[...])
        ct = bmm(g2[...], g3[...])
        tr = jnp.sum(a * ct, axis=(1, 2))
        out = jnp.log(jnp.clip(tr, 1e-12)) - lognorm[0, 0]
        out_ref[0, sub] = out


def kernel(index, log_core_0, log_core_1, log_core_2, log_core_3,
           log_core_4, log_core_5, log_core_6, log_core_7):
    offs = jnp.array([0, 64, 128, 192, 0, 64, 128, 192], dtype=jnp.int32)
    idx_flat = index + offs[None, :]

    core_spec = pl.BlockSpec((_R, _R, _R), lambda i: (0, 0, 0))
    scratch = (
        [pltpu.VMEM((4 * _R, _R, _R), jnp.bfloat16) for _ in range(2)]
        + [pltpu.VMEM((_TILE, _R, _R), jnp.bfloat16) for _ in range(10)]
        + [pltpu.VMEM((_R, _R, _R), jnp.float32),
           pltpu.VMEM((_R, _R), jnp.float32),
           pltpu.SMEM((1, 1), jnp.float32)]
    )

    out = pl.pallas_call(
        _tr_kernel,
        grid=(_NT,),
        in_specs=[pl.BlockSpec((_STEP, _NC), lambda i: (i, 0),
                               memory_space=pltpu.SMEM)] + [core_spec] * 8,
        out_specs=pl.BlockSpec((1, _SUB, _TILE), lambda i: (i, 0, 0)),
        out_shape=jax.ShapeDtypeStruct((_NT, _SUB, _TILE), jnp.float32),
        scratch_shapes=scratch,
        compiler_params=pltpu.CompilerParams(
            dimension_semantics=("arbitrary",),
        ),
    )(idx_flat, log_core_0, log_core_1, log_core_2, log_core_3,
      log_core_4, log_core_5, log_core_6, log_core_7)

    return out.reshape(_B)


# speedup vs baseline: 1.1095x; 1.1095x over previous
import jax
import jax.numpy as jnp
from jax.experimental import pallas as pl
from jax.experimental.pallas import tpu as pltpu

_B = 4096
_R = 64
_NC = 8
_TILE = 64
_SUB = 8
_STEP = _TILE * _SUB
_NT = _B // _STEP

_BMM_DIMS = (((2,), (1,)), ((0,), (0,)))


def _tr_kernel(idx_ref, c0_ref, c1_ref, c2_ref, c3_ref, c4_ref, c5_ref,
               c6_ref, c7_ref, out_ref,
               tbl, tblt, g0, g1, g2, g3, g4, g5, g6, g7, s0, s1,
               norm, lognorm):
    @pl.when(pl.program_id(0) == 0)
    def _build_tables():
        crefs = [c0_ref, c1_ref, c2_ref, c3_ref, c4_ref, c5_ref, c6_ref,
                 c7_ref]
        for i in range(_NC):
            g0[...] = jax.nn.softplus(crefs[i][...])
            if i < 4:
                tbl[pl.ds(_R * i, _R)] = g0[...]
            else:
                tblt[pl.ds(_R * (i - 4), _R)] = jnp.swapaxes(g0[...], 1, 2)
            s = jnp.sum(g0[...], axis=0)
            if i == 0:
                norm[...] = s
            else:
                norm[...] = norm[...] @ s
        eye = (jax.lax.broadcasted_iota(jnp.int32, (_R, _R), 0)
               == jax.lax.broadcasted_iota(jnp.int32, (_R, _R), 1))
        tr_n = jnp.sum(jnp.where(eye, norm[...], 0.0))
        lognorm[0, 0] = jnp.log(tr_n)

    def bmm(x, y):
        return jax.lax.dot_general(x, y, _BMM_DIMS,
                                   preferred_element_type=jnp.float32)

    for sub in range(_SUB):
        base = sub * _TILE

        def gather_body(b, carry):
            r = base + b
            g0[b] = tbl[idx_ref[r, 0]]
            g1[b] = tbl[idx_ref[r, 1]]
            g2[b] = tbl[idx_ref[r, 2]]
            g3[b] = tbl[idx_ref[r, 3]]
            g4[b] = tblt[idx_ref[r, 4]]
            g5[b] = tblt[idx_ref[r, 5]]
            g6[b] = tblt[idx_ref[r, 6]]
            g7[b] = tblt[idx_ref[r, 7]]
            return carry

        jax.lax.fori_loop(0, _TILE, gather_body, 0, unroll=8)

        s0[...] = bmm(g0[...], g1[...])
        s1[...] = bmm(g2[...], g3[...])
        g2[...] = bmm(g7[...], g6[...])
        g3[...] = bmm(g5[...], g4[...])
        g0[...] = bmm(s0[...], s1[...])
        g1[...] = bmm(g2[...], g3[...])
        tr = jnp.sum(g0[...] * g1[...], axis=(1, 2))
        out = jnp.log(jnp.clip(tr, 1e-12)) - lognorm[0, 0]
        out_ref[0, sub] = out


def kernel(index, log_core_0, log_core_1, log_core_2, log_core_3,
           log_core_4, log_core_5, log_core_6, log_core_7):
    offs = jnp.array([0, 64, 128, 192, 0, 64, 128, 192], dtype=jnp.int32)
    idx_flat = index + offs[None, :]

    core_spec = pl.BlockSpec((_R, _R, _R), lambda i: (0, 0, 0))
    scratch = (
        [pltpu.VMEM((4 * _R, _R, _R), jnp.float32) for _ in range(2)]
        + [pltpu.VMEM((_TILE, _R, _R), jnp.float32) for _ in range(10)]
        + [pltpu.VMEM((_R, _R), jnp.float32),
           pltpu.SMEM((1, 1), jnp.float32)]
    )

    out = pl.pallas_call(
        _tr_kernel,
        grid=(_NT,),
        in_specs=[pl.BlockSpec((_STEP, _NC), lambda i: (i, 0),
                               memory_space=pltpu.SMEM)] + [core_spec] * 8,
        out_specs=pl.BlockSpec((1, _SUB, _TILE), lambda i: (i, 0, 0)),
        out_shape=jax.ShapeDtypeStruct((_NT, _SUB, _TILE), jnp.float32),
        scratch_shapes=scratch,
        compiler_params=pltpu.CompilerParams(
            dimension_semantics=("arbitrary",),
        ),
    )(idx_flat, log_core_0, log_core_1, log_core_2, log_core_3,
      log_core_4, log_core_5, log_core_6, log_core_7)

    return out.reshape(_B)
